# baseline (speedup 1.0000x reference)
import jax
import jax.numpy as jnp
from jax.experimental import pallas as pl
from jax.experimental.pallas import tpu as pltpu

_STRIDE = 8.0
_NC = 80
_ROWS = 512
_POS_STEP = _ROWS * 8


def _rot(x, s):
    return jnp.roll(x, -s, axis=1)


def _detect_body(anchors_ref, cls_ref, bbox_ref, conf_ref,
                 pb_ref, idx_ref, confs_ref):
    t = pl.program_id(0)

    x = cls_ref[...]
    lpos = jax.lax.broadcasted_iota(jnp.int32, x.shape, 1) % _NC
    mval = x
    midx = lpos
    for s in (1, 2, 1, 5, 10, 20, 40):
        cv = _rot(mval, s)
        ci = _rot(midx, s)
        ok = (lpos < (_NC - s)) & (cv > mval)
        mval = jnp.where(ok, cv, mval)
        midx = jnp.where(ok, ci, midx)
    m8 = jnp.concatenate([mval[:, 80 * k:80 * k + 1] for k in range(8)],
                         axis=1)
    i8 = jnp.concatenate([midx[:, 80 * k:80 * k + 1] for k in range(8)],
                         axis=1)
    idx_ref[...] = i8
    confs_ref[...] = jax.nn.sigmoid(conf_ref[...]) * jax.nn.sigmoid(m8)

    bb = bbox_ref[...]
    s4 = jax.nn.sigmoid(bb)
    rowi = jax.lax.broadcasted_iota(jnp.int32, bb.shape, 0) + t * bb.shape[0]
    lane = jax.lax.broadcasted_iota(jnp.int32, bb.shape, 1)
    flat4 = rowi * 1024 + lane
    ch = lane % 4
    pos = flat4 // 4
    w = (pos % 64).astype(jnp.float32)
    h = ((pos // 64) % 64).astype(jnp.float32)
    a = (pos // 4096) % 3
    xy = (s4 * 2.0 - 0.5 + jnp.where(ch == 0, w, h)) * _STRIDE
    aw = jnp.where(a == 0, anchors_ref[0, 0],
                   jnp.where(a == 1, anchors_ref[1, 0], anchors_ref[2, 0]))
    ah = jnp.where(a == 0, anchors_ref[0, 1],
                   jnp.where(a == 1, anchors_ref[1, 1], anchors_ref[2, 1]))
    wh = (s4 * 2.0) ** 2 * jnp.where(ch == 2, aw, ah)
    pb_ref[...] = jnp.where(ch < 2, xy, wh)


def kernel(bbox, conf, cls_logits, anchors):
    nB, nA, nH, nW, nC = cls_logits.shape
    P = nH * nW
    n = nA * P
    NPOS = nB * n
    R = NPOS * nC // 640
    RB = NPOS * 4 // 1024
    R8 = NPOS // 8

    steps = R // _ROWS
    rb = RB // steps
    pb, idx, confs = pl.pallas_call(
        _detect_body,
        grid=(steps,),
        in_specs=[
            pl.BlockSpec(memory_space=pltpu.SMEM),
            pl.BlockSpec((_ROWS, 640), lambda t: (t, 0)),
            pl.BlockSpec((rb, 1024), lambda t: (t, 0)),
            pl.BlockSpec((_ROWS, 8), lambda t: (t, 0)),
        ],
        out_specs=[
            pl.BlockSpec((rb, 1024), lambda t: (t, 0)),
            pl.BlockSpec((_ROWS, 8), lambda t: (t, 0)),
            pl.BlockSpec((_ROWS, 8), lambda t: (t, 0)),
        ],
        out_shape=[
            jax.ShapeDtypeStruct((RB, 1024), jnp.float32),
            jax.ShapeDtypeStruct((R8, 8), jnp.int32),
            jax.ShapeDtypeStruct((R8, 8), jnp.float32),
        ],
        compiler_params=pltpu.CompilerParams(
            dimension_semantics=("arbitrary",)),
    )(anchors, cls_logits.reshape(R, 640), bbox.reshape(RB, 1024),
      conf.reshape(R8, 8))

    return (pb.reshape(nB, n, 4), idx.reshape(nB, n), confs.reshape(nB, n))


# speedup vs baseline: 1.4843x; 1.4843x over previous
import functools

import jax
import jax.numpy as jnp
from jax import lax
from jax.experimental import pallas as pl
from jax.experimental.pallas import tpu as pltpu
from jax.experimental.pallas import tpu_sc as plsc

_STRIDE = 8.0
_NC = 80
_NPOS = 196608
_NW = 32
_PER_W = _NPOS // _NW
_CHUNK = 256
_NCH = _PER_W // _CHUNK


def _sc_cls_head(cls_hbm, m_hbm, idx_hbm, shared, buf0, buf1, mout0, iout0,
                 mout1, iout1, sem0, sem1, osem0, osem1):
    sid = lax.axis_index("s")
    wid = sid * 2 + lax.axis_index("c")
    base = wid * _PER_W
    sems = (sem0, sem1)
    bufs = (buf0, buf1)
    mouts = (mout0, mout1)
    iouts = (iout0, iout1)
    osems = (osem0, osem1)
    lane80 = lax.iota(jnp.int32, 16) * _NC

    def in_desc(g, b):
        src = cls_hbm.at[pl.ds((base + g * _CHUNK) * _NC, _CHUNK * _NC)]
        return pltpu.make_async_copy(src, shared.at[b, sid], sems[b])

    def out_descs(g, b):
        dst = pl.ds(base + g * _CHUNK, _CHUNK)
        return (pltpu.make_async_copy(mouts[b], m_hbm.at[dst], osems[b]),
                pltpu.make_async_copy(iouts[b], idx_hbm.at[dst], osems[b]))

    in_desc(0, 0).start()
    in_desc(1, 1).start()

    def chunk_pair(j, carry):
        for b in (0, 1):
            g = j * 2 + b
            in_desc(g, b).wait()
            pltpu.sync_copy(shared.at[b, sid], bufs[b])

            @pl.when(g + 2 < _NCH)
            def _():
                in_desc(g + 2, b).start()

            @pl.when(g >= 2)
            def _():
                for d in out_descs(g - 2, b):
                    d.wait()
            bufb, moutb, ioutb = bufs[b], mouts[b], iouts[b]

            @plsc.parallel_loop(0, _CHUNK // 16, unroll=2)
            def group_body(g2, bufb=bufb, moutb=moutb, ioutb=ioutb):
                idx0 = g2 * (16 * _NC) + lane80
                mx = [plsc.load_gather(bufb, [idx0 + c]) for c in range(4)]
                am = [jnp.full((16,), c, jnp.int32) for c in range(4)]
                for c in range(4, _NC):
                    q = c % 4
                    v = plsc.load_gather(bufb, [idx0 + c])
                    gt = v > mx[q]
                    am[q] = jnp.where(gt, c, am[q])
                    mx[q] = jnp.maximum(mx[q], v)

                def merge(m0, a0, m1, a1):
                    take1 = (m1 > m0) | ((m1 == m0) & (a1 < a0))
                    return (jnp.where(take1, m1, m0),
                            jnp.where(take1, a1, a0))

                m01, a01 = merge(mx[0], am[0], mx[1], am[1])
                m23, a23 = merge(mx[2], am[2], mx[3], am[3])
                m, a = merge(m01, a01, m23, a23)
                moutb[pl.ds(g2 * 16, 16)] = m
                ioutb[pl.ds(g2 * 16, 16)] = a

            for d in out_descs(g, b):
                d.start()
        return carry

    lax.fori_loop(0, _NCH // 2, chunk_pair, 0)
    for g in (_NCH - 2, _NCH - 1):
        for d in out_descs(g, g % 2):
            d.wait()


def _tc_decode(anchors_ref, bbox_ref, conf_ref, m_ref, pb_ref, confs_ref):
    i = pl.program_id(0)

    bb = bbox_ref[...]
    s4 = jax.nn.sigmoid(bb)
    rowi = jax.lax.broadcasted_iota(jnp.int32, bb.shape, 0) + i * bb.shape[0]
    lane = jax.lax.broadcasted_iota(jnp.int32, bb.shape, 1)
    flat4 = rowi * 1024 + lane
    ch = lane % 4
    pos = flat4 // 4
    w = (pos % 64).astype(jnp.float32)
    h = ((pos // 64) % 64).astype(jnp.float32)
    a = (pos // 4096) % 3
    xy = (s4 * 2.0 - 0.5 + jnp.where(ch == 0, w, h)) * _STRIDE
    aw = jnp.where(a == 0, anchors_ref[0, 0],
                   jnp.where(a == 1, anchors_ref[1, 0], anchors_ref[2, 0]))
    ah = jnp.where(a == 0, anchors_ref[0, 1],
                   jnp.where(a == 1, anchors_ref[1, 1], anchors_ref[2, 1]))
    wh = (s4 * 2.0) ** 2 * jnp.where(ch == 2, aw, ah)
    pb_ref[...] = jnp.where(ch < 2, xy, wh)

    confs_ref[...] = jax.nn.sigmoid(conf_ref[...]) * jax.nn.sigmoid(m_ref[...])


def kernel(bbox, conf, cls_logits, anchors):
    nB, nA, nH, nW, nC = cls_logits.shape
    P = nH * nW
    n = nA * P

    sc_call = functools.partial(
        pl.kernel,
        out_type=[
            jax.ShapeDtypeStruct((_NPOS,), jnp.float32),
            jax.ShapeDtypeStruct((_NPOS,), jnp.int32),
        ],
        mesh=plsc.VectorSubcoreMesh(
            core_axis_name="c", subcore_axis_name="s",
            num_cores=2, num_subcores=16),
        compiler_params=pltpu.CompilerParams(needs_layout_passes=False),
        scratch_types=[
            pltpu.VMEM_SHARED((2, 16, _CHUNK * _NC), jnp.float32),
            pltpu.VMEM((_CHUNK * _NC,), jnp.float32),
            pltpu.VMEM((_CHUNK * _NC,), jnp.float32),
            pltpu.VMEM((_CHUNK,), jnp.float32),
            pltpu.VMEM((_CHUNK,), jnp.int32),
            pltpu.VMEM((_CHUNK,), jnp.float32),
            pltpu.VMEM((_CHUNK,), jnp.int32),
            pltpu.SemaphoreType.DMA,
            pltpu.SemaphoreType.DMA,
            pltpu.SemaphoreType.DMA,
            pltpu.SemaphoreType.DMA,
        ],
    )(_sc_cls_head)
    m_flat, idx_flat = sc_call(cls_logits.reshape(_NPOS * _NC))

    RB = _NPOS * 4 // 1024
    RC = _NPOS // 1024
    gsteps = 4
    pb, confs = pl.pallas_call(
        _tc_decode,
        grid=(gsteps,),
        in_specs=[
            pl.BlockSpec(memory_space=pltpu.SMEM),
            pl.BlockSpec((RB // gsteps, 1024), lambda k: (k, 0)),
            pl.BlockSpec((RC // gsteps, 1024), lambda k: (k, 0)),
            pl.BlockSpec((RC // gsteps, 1024), lambda k: (k, 0)),
        ],
        out_specs=[
            pl.BlockSpec((RB // gsteps, 1024), lambda k: (k, 0)),
            pl.BlockSpec((RC // gsteps, 1024), lambda k: (k, 0)),
        ],
        out_shape=[
            jax.ShapeDtypeStruct((RB, 1024), jnp.float32),
            jax.ShapeDtypeStruct((RC, 1024), jnp.float32),
        ],
        compiler_params=pltpu.CompilerParams(
            dimension_semantics=("arbitrary",)),
    )(anchors, bbox.reshape(RB, 1024), conf.reshape(RC, 1024),
      m_flat.reshape(RC, 1024))

    return (pb.reshape(nB, n, 4), idx_flat.reshape(nB, n),
            confs.reshape(nB, n))


# speedup vs baseline: 1.6134x; 1.0870x over previous
import functools

import jax
import jax.numpy as jnp
from jax import lax
from jax.experimental import pallas as pl
from jax.experimental.pallas import tpu as pltpu
from jax.experimental.pallas import tpu_sc as plsc

_STRIDE = 8.0
_NC = 80
_NPOS = 196608
_NW = 32
_PER_W = _NPOS // _NW
_CHUNK = 512
_NCH = _PER_W // _CHUNK


def _sc_cls_head(cls_hbm, m_hbm, idx_hbm, buf0, buf1, mout0, iout0,
                 mout1, iout1, sem0, sem1, osem0, osem1):
    wid = lax.axis_index("s") * 2 + lax.axis_index("c")
    base = wid * _PER_W
    sems = (sem0, sem1)
    bufs = (buf0, buf1)
    mouts = (mout0, mout1)
    iouts = (iout0, iout1)
    osems = (osem0, osem1)
    lane80 = lax.iota(jnp.int32, 16) * _NC

    def start(g):
        b = g % 2
        src = cls_hbm.at[pl.ds((base + g * _CHUNK) * _NC, _CHUNK * _NC)]
        return pltpu.async_copy(src, bufs[b], sems[b])

    handles = {0: start(0)}
    out_handles = {}
    for g in range(_NCH):
        b = g % 2
        if g + 1 < _NCH:
            handles[g + 1] = start(g + 1)
        handles.pop(g).wait()
        if g - 2 in out_handles:
            for h in out_handles.pop(g - 2):
                h.wait()
        bufb, moutb, ioutb = bufs[b], mouts[b], iouts[b]

        @plsc.parallel_loop(0, _CHUNK // 16, unroll=2)
        def group_body(g2, bufb=bufb, moutb=moutb, ioutb=ioutb):
            idx0 = g2 * (16 * _NC) + lane80
            mx = [plsc.load_gather(bufb, [idx0 + c]) for c in range(4)]
            am = [jnp.full((16,), c, jnp.int32) for c in range(4)]
            for c in range(4, _NC):
                q = c % 4
                v = plsc.load_gather(bufb, [idx0 + c])
                gt = v > mx[q]
                am[q] = jnp.where(gt, c, am[q])
                mx[q] = jnp.maximum(mx[q], v)

            def merge(m0, a0, m1, a1):
                take1 = (m1 > m0) | ((m1 == m0) & (a1 < a0))
                return (jnp.where(take1, m1, m0), jnp.where(take1, a1, a0))

            m01, a01 = merge(mx[0], am[0], mx[1], am[1])
            m23, a23 = merge(mx[2], am[2], mx[3], am[3])
            m, a = merge(m01, a01, m23, a23)
            moutb[pl.ds(g2 * 16, 16)] = m
            ioutb[pl.ds(g2 * 16, 16)] = a

        dst = pl.ds(base + g * _CHUNK, _CHUNK)
        out_handles[g] = (pltpu.async_copy(moutb, m_hbm.at[dst], osems[b]),
                          pltpu.async_copy(ioutb, idx_hbm.at[dst], osems[b]))
    for hs in out_handles.values():
        for h in hs:
            h.wait()


def _tc_decode(anchors_ref, bbox_ref, conf_ref, m_ref, pb_ref, confs_ref):
    i = pl.program_id(0)

    bb = bbox_ref[...]
    s4 = jax.nn.sigmoid(bb)
    rowi = jax.lax.broadcasted_iota(jnp.int32, bb.shape, 0) + i * bb.shape[0]
    lane = jax.lax.broadcasted_iota(jnp.int32, bb.shape, 1)
    flat4 = rowi * 1024 + lane
    ch = lane % 4
    pos = flat4 // 4
    w = (pos % 64).astype(jnp.float32)
    h = ((pos // 64) % 64).astype(jnp.float32)
    a = (pos // 4096) % 3
    xy = (s4 * 2.0 - 0.5 + jnp.where(ch == 0, w, h)) * _STRIDE
    aw = jnp.where(a == 0, anchors_ref[0, 0],
                   jnp.where(a == 1, anchors_ref[1, 0], anchors_ref[2, 0]))
    ah = jnp.where(a == 0, anchors_ref[0, 1],
                   jnp.where(a == 1, anchors_ref[1, 1], anchors_ref[2, 1]))
    wh = (s4 * 2.0) ** 2 * jnp.where(ch == 2, aw, ah)
    pb_ref[...] = jnp.where(ch < 2, xy, wh)

    confs_ref[...] = jax.nn.sigmoid(conf_ref[...]) * jax.nn.sigmoid(m_ref[...])


def kernel(bbox, conf, cls_logits, anchors):
    nB, nA, nH, nW, nC = cls_logits.shape
    P = nH * nW
    n = nA * P

    sc_call = functools.partial(
        pl.kernel,
        out_type=[
            jax.ShapeDtypeStruct((_NPOS,), jnp.float32),
            jax.ShapeDtypeStruct((_NPOS,), jnp.int32),
        ],
        mesh=plsc.VectorSubcoreMesh(
            core_axis_name="c", subcore_axis_name="s",
            num_cores=2, num_subcores=16),
        compiler_params=pltpu.CompilerParams(needs_layout_passes=False),
        scratch_types=[
            pltpu.VMEM((_CHUNK * _NC,), jnp.float32),
            pltpu.VMEM((_CHUNK * _NC,), jnp.float32),
            pltpu.VMEM((_CHUNK,), jnp.float32),
            pltpu.VMEM((_CHUNK,), jnp.int32),
            pltpu.VMEM((_CHUNK,), jnp.float32),
            pltpu.VMEM((_CHUNK,), jnp.int32),
            pltpu.SemaphoreType.DMA,
            pltpu.SemaphoreType.DMA,
            pltpu.SemaphoreType.DMA,
            pltpu.SemaphoreType.DMA,
        ],
    )(_sc_cls_head)
    m_flat, idx_flat = sc_call(cls_logits.reshape(_NPOS * _NC))

    RB = _NPOS * 4 // 1024
    RC = _NPOS // 1024
    gsteps = 4
    pb, confs = pl.pallas_call(
        _tc_decode,
        grid=(gsteps,),
        in_specs=[
            pl.BlockSpec(memory_space=pltpu.SMEM),
            pl.BlockSpec((RB // gsteps, 1024), lambda k: (k, 0)),
            pl.BlockSpec((RC // gsteps, 1024), lambda k: (k, 0)),
            pl.BlockSpec((RC // gsteps, 1024), lambda k: (k, 0)),
        ],
        out_specs=[
            pl.BlockSpec((RB // gsteps, 1024), lambda k: (k, 0)),
            pl.BlockSpec((RC // gsteps, 1024), lambda k: (k, 0)),
        ],
        out_shape=[
            jax.ShapeDtypeStruct((RB, 1024), jnp.float32),
            jax.ShapeDtypeStruct((RC, 1024), jnp.float32),
        ],
        compiler_params=pltpu.CompilerParams(
            dimension_semantics=("arbitrary",)),
    )(anchors, bbox.reshape(RB, 1024), conf.reshape(RC, 1024),
      m_flat.reshape(RC, 1024))

    return (pb.reshape(nB, n, 4), idx_flat.reshape(nB, n),
            confs.reshape(nB, n))
